# baseline (speedup 1.0000x reference)
import functools

import jax
import jax.numpy as jnp
from jax.experimental import pallas as pl
from jax.experimental.pallas import tpu as pltpu

_NSLOTS = 3
_NOUT = 3
_LOOKAHEAD = 2


def _se_manual(x_hbm, w1_ref, w2_ref, o_hbm, ibuf, obuf, in_sem, out_sem):
    NB, R, S = x_hbm.shape
    C = R // 2
    inv_S = 1.0 / float(S)
    w1 = w1_ref[...]
    w2 = w2_ref[...]

    def start_in(b):
        pltpu.make_async_copy(
            x_hbm.at[b], ibuf.at[b % _NSLOTS], in_sem.at[b % _NSLOTS]
        ).start()

    def wait_in(b):
        pltpu.make_async_copy(
            x_hbm.at[0], ibuf.at[b % _NSLOTS], in_sem.at[b % _NSLOTS]
        ).wait()

    def start_out(b):
        pltpu.make_async_copy(
            obuf.at[b % _NOUT], o_hbm.at[b], out_sem.at[b % _NOUT]
        ).start()

    def wait_out(b):
        pltpu.make_async_copy(
            obuf.at[b % _NOUT], o_hbm.at[0], out_sem.at[b % _NOUT]
        ).wait()

    for b in range(_LOOKAHEAD):
        start_in(b)
    for b in range(NB):
        wait_in(b)
        if b + _LOOKAHEAD < NB:
            start_in(b + _LOOKAHEAD)
        xb = ibuf[b % _NSLOTS]
        mean = jnp.sum(xb, axis=-1, keepdims=True,
                       dtype=jnp.float32) * inv_S
        m2 = jnp.concatenate([mean[:C], mean[C:]], axis=1)
        h = jnp.maximum(
            jnp.dot(w1, m2, preferred_element_type=jnp.float32), 0.0)
        g = jnp.dot(w2, h, preferred_element_type=jnp.float32)
        gate = 1.0 / (1.0 + jnp.exp(-g))
        gate2 = jnp.concatenate([gate[:, :1], gate[:, 1:2]], axis=0)
        if b >= _NOUT:
            wait_out(b - _NOUT)
        obuf[b % _NOUT] = xb * gate2
        start_out(b)
    for b in range(max(0, NB - _NOUT), NB):
        wait_out(b)


@jax.jit
def _se3d(x, w1, w2):
    B, C, D, H, W = x.shape
    S = D * H * W
    x3 = x.reshape(B // 2, 2 * C, S)
    out = pl.pallas_call(
        _se_manual,
        out_shape=jax.ShapeDtypeStruct((B // 2, 2 * C, S), x.dtype),
        in_specs=[
            pl.BlockSpec(memory_space=pltpu.MemorySpace.HBM),
            pl.BlockSpec(memory_space=pltpu.MemorySpace.VMEM),
            pl.BlockSpec(memory_space=pltpu.MemorySpace.VMEM),
        ],
        out_specs=pl.BlockSpec(memory_space=pltpu.MemorySpace.HBM),
        scratch_shapes=[
            pltpu.VMEM((_NSLOTS, 2 * C, S), jnp.float32),
            pltpu.VMEM((_NOUT, 2 * C, S), jnp.float32),
            pltpu.SemaphoreType.DMA((_NSLOTS,)),
            pltpu.SemaphoreType.DMA((_NOUT,)),
        ],
        compiler_params=pltpu.CompilerParams(
            vmem_limit_bytes=58 * 1024 * 1024,
        ),
    )(x3, w1, w2)
    return out.reshape(B, C, D, H, W)


def kernel(x, w1, w2):
    return _se3d(x, w1, w2)


# speedup vs baseline: 2.5219x; 2.5219x over previous
import functools

import jax
import jax.numpy as jnp
from jax.experimental import pallas as pl
from jax.experimental.pallas import tpu as pltpu

_NSLOTS = 6
_NOUT = 5
_LOOKAHEAD = 4


def _se_manual(x_hbm, w1_ref, w2_ref, o_hbm, ibuf, obuf, in_sem, out_sem):
    B, C, S = x_hbm.shape
    inv_S = 1.0 / float(S)
    w1 = w1_ref[...]
    w2 = w2_ref[...]

    def start_in(b):
        pltpu.make_async_copy(
            x_hbm.at[b], ibuf.at[b % _NSLOTS], in_sem.at[b % _NSLOTS]
        ).start()

    def wait_in(b):
        pltpu.make_async_copy(
            x_hbm.at[0], ibuf.at[b % _NSLOTS], in_sem.at[b % _NSLOTS]
        ).wait()

    def start_out(b):
        pltpu.make_async_copy(
            obuf.at[b % _NOUT], o_hbm.at[b], out_sem.at[b % _NOUT]
        ).start()

    def wait_out(b):
        pltpu.make_async_copy(
            obuf.at[b % _NOUT], o_hbm.at[0], out_sem.at[b % _NOUT]
        ).wait()

    for b in range(_LOOKAHEAD):
        start_in(b)
    for b in range(B):
        wait_in(b)
        if b + _LOOKAHEAD < B:
            start_in(b + _LOOKAHEAD)
        xb = ibuf[b % _NSLOTS]
        mean = jnp.sum(xb, axis=-1, keepdims=True,
                       dtype=jnp.float32) * inv_S
        h = jnp.maximum(
            jnp.dot(w1, mean, preferred_element_type=jnp.float32), 0.0)
        g = jnp.dot(w2, h, preferred_element_type=jnp.float32)
        gate = 1.0 / (1.0 + jnp.exp(-g))
        if b >= _NOUT:
            wait_out(b - _NOUT)
        obuf[b % _NOUT] = xb * gate
        start_out(b)
    for b in range(B - _NOUT, B):
        wait_out(b)


@jax.jit
def _se3d(x, w1, w2):
    B, C, D, H, W = x.shape
    S = D * H * W
    x3 = x.reshape(B, C, S)
    out = pl.pallas_call(
        _se_manual,
        out_shape=jax.ShapeDtypeStruct((B, C, S), x.dtype),
        in_specs=[
            pl.BlockSpec(memory_space=pltpu.MemorySpace.HBM),
            pl.BlockSpec(memory_space=pltpu.MemorySpace.VMEM),
            pl.BlockSpec(memory_space=pltpu.MemorySpace.VMEM),
        ],
        out_specs=pl.BlockSpec(memory_space=pltpu.MemorySpace.HBM),
        scratch_shapes=[
            pltpu.VMEM((_NSLOTS, C, S), jnp.float32),
            pltpu.VMEM((_NOUT, C, S), jnp.float32),
            pltpu.SemaphoreType.DMA((_NSLOTS,)),
            pltpu.SemaphoreType.DMA((_NOUT,)),
        ],
        compiler_params=pltpu.CompilerParams(
            vmem_limit_bytes=50 * 1024 * 1024,
        ),
    )(x3, w1, w2)
    return out.reshape(B, C, D, H, W)


def kernel(x, w1, w2):
    return _se3d(x, w1, w2)


# speedup vs baseline: 2.5333x; 1.0045x over previous
import jax
import jax.numpy as jnp
from jax.experimental import pallas as pl
from jax.experimental.pallas import tpu as pltpu

_NSLOTS = 6
_NOUT = 5
_LOOKAHEAD = 4


def _se_manual(x_hbm, w1_ref, w2_ref, o_hbm, ibuf, obuf, in_sem, out_sem):
    B, C, S = x_hbm.shape
    inv_S = 1.0 / float(S)
    w1 = w1_ref[...]
    w2 = w2_ref[...]

    def start_in(b):
        pltpu.make_async_copy(
            x_hbm.at[b], ibuf.at[b % _NSLOTS], in_sem.at[b % _NSLOTS]
        ).start()

    def wait_in(b):
        pltpu.make_async_copy(
            x_hbm.at[0], ibuf.at[b % _NSLOTS], in_sem.at[b % _NSLOTS]
        ).wait()

    def start_out(b):
        pltpu.make_async_copy(
            obuf.at[b % _NOUT], o_hbm.at[b], out_sem.at[b % _NOUT]
        ).start()

    def wait_out(b):
        pltpu.make_async_copy(
            obuf.at[b % _NOUT], o_hbm.at[0], out_sem.at[b % _NOUT]
        ).wait()

    for b in range(min(_LOOKAHEAD, B)):
        start_in(b)
    for b in range(B):
        wait_in(b)
        if b + _LOOKAHEAD < B:
            start_in(b + _LOOKAHEAD)
        xb = ibuf[b % _NSLOTS]
        mean = jnp.sum(xb, axis=-1, keepdims=True,
                       dtype=jnp.float32) * inv_S
        h = jnp.maximum(
            jnp.dot(w1, mean, preferred_element_type=jnp.float32), 0.0)
        g = jnp.dot(w2, h, preferred_element_type=jnp.float32)
        gate = 1.0 / (1.0 + jnp.exp(-g))
        if b >= _NOUT:
            wait_out(b - _NOUT)
        obuf[b % _NOUT] = xb * gate
        start_out(b)
    for b in range(max(0, B - _NOUT), B):
        wait_out(b)


@jax.jit
def _se3d(x, w1, w2):
    B, C, D, H, W = x.shape
    S = D * H * W
    x3 = x.reshape(B, C, S)
    out = pl.pallas_call(
        _se_manual,
        out_shape=jax.ShapeDtypeStruct((B, C, S), x.dtype),
        in_specs=[
            pl.BlockSpec(memory_space=pltpu.MemorySpace.HBM),
            pl.BlockSpec(memory_space=pltpu.MemorySpace.VMEM),
            pl.BlockSpec(memory_space=pltpu.MemorySpace.VMEM),
        ],
        out_specs=pl.BlockSpec(memory_space=pltpu.MemorySpace.HBM),
        scratch_shapes=[
            pltpu.VMEM((_NSLOTS, C, S), jnp.float32),
            pltpu.VMEM((_NOUT, C, S), jnp.float32),
            pltpu.SemaphoreType.DMA((_NSLOTS,)),
            pltpu.SemaphoreType.DMA((_NOUT,)),
        ],
        compiler_params=pltpu.CompilerParams(
            vmem_limit_bytes=50 * 1024 * 1024,
        ),
    )(x3, w1, w2)
    return out.reshape(B, C, D, H, W)


def kernel(x, w1, w2):
    return _se3d(x, w1, w2)
